# Initial kernel scaffold; baseline (speedup 1.0000x reference)
import jax
import jax.numpy as jnp
from jax.experimental import pallas as pl


def kernel(x, adj, Ws, As):
    raise NotImplementedError("write your pallas kernel here")


# speedup vs baseline: 1.7166x; 1.7166x over previous
import functools

import jax
import jax.numpy as jnp
from jax import lax
from jax.experimental import pallas as pl
from jax.experimental.pallas import tpu as pltpu

_N = 4096
_NFEAT = 256
_NHID = 16
_NHEADS = 4
_ALPHA = 0.2
_BLK = 512


def _wh_body(x_ref, w_ref, wh_ref):
    wh_ref[0] = jnp.dot(x_ref[...], w_ref[0], preferred_element_type=jnp.float32)


def _gat_body(adj_ref, wh_ref, a_ref, out_ref):
    i = pl.program_id(0)
    adj = adj_ref[...]
    mask = adj > 0.0
    for h in range(_NHEADS):
        wh = wh_ref[h]
        whb = wh_ref[h, pl.ds(i * _BLK, _BLK), :]
        a1 = a_ref[h, :_NHID, :]
        a2 = a_ref[h, _NHID:, :]
        f1 = lax.dot_general(whb, a1, (((1,), (0,)), ((), ())),
                             preferred_element_type=jnp.float32)
        f2 = lax.dot_general(a2, wh, (((0,), (1,)), ((), ())),
                             preferred_element_type=jnp.float32)
        e = f1 + f2
        e = jnp.where(e >= 0.0, e, _ALPHA * e)
        att = jnp.where(mask, e, -9e15)
        m = jnp.max(att, axis=1, keepdims=True)
        p = jnp.exp(att - m)
        s = jnp.sum(p, axis=1, keepdims=True)
        attn = p / s
        hp = jnp.dot(attn, wh, preferred_element_type=jnp.float32)
        out_ref[:, h * _NHID:(h + 1) * _NHID] = jnp.where(
            hp > 0.0, hp, jnp.exp(hp) - 1.0)


@jax.jit
def kernel(x, adj, Ws, As):
    wh = pl.pallas_call(
        _wh_body,
        grid=(_NHEADS,),
        in_specs=[
            pl.BlockSpec((_N, _NFEAT), lambda h: (0, 0)),
            pl.BlockSpec((1, _NFEAT, _NHID), lambda h: (h, 0, 0)),
        ],
        out_specs=pl.BlockSpec((1, _N, _NHID), lambda h: (h, 0, 0)),
        out_shape=jax.ShapeDtypeStruct((_NHEADS, _N, _NHID), jnp.float32),
    )(x, Ws)

    out = pl.pallas_call(
        _gat_body,
        grid=(_N // _BLK,),
        in_specs=[
            pl.BlockSpec((_BLK, _N), lambda i: (i, 0)),
            pl.BlockSpec((_NHEADS, _N, _NHID), lambda i: (0, 0, 0)),
            pl.BlockSpec((_NHEADS, 2 * _NHID, 1), lambda i: (0, 0, 0)),
        ],
        out_specs=pl.BlockSpec((_BLK, _NHEADS * _NHID), lambda i: (i, 0)),
        out_shape=jax.ShapeDtypeStruct((_N, _NHEADS * _NHID), jnp.float32),
    )(adj, wh, As)
    return out


# speedup vs baseline: 2.0747x; 1.2086x over previous
import functools

import jax
import jax.numpy as jnp
from jax import lax
from jax.experimental import pallas as pl
from jax.experimental.pallas import tpu as pltpu

_N = 4096
_NFEAT = 256
_NHID = 16
_NHEADS = 4
_ALPHA = 0.2
_BLK = 512


def _wh_body(x_ref, w_ref, wh_ref):
    wh_ref[0] = jnp.dot(x_ref[...], w_ref[0], preferred_element_type=jnp.float32)


def _gat_body(adj_ref, wh_ref, a_ref, out_ref):
    i = pl.program_id(0)
    adj = adj_ref[...]
    for h in range(_NHEADS):
        wh = wh_ref[h]
        whb = wh_ref[h, pl.ds(i * _BLK, _BLK), :]
        a1 = a_ref[h, :_NHID, :]
        a2 = a_ref[h, _NHID:, :]
        f1 = lax.dot_general(whb, a1, (((1,), (0,)), ((), ())),
                             preferred_element_type=jnp.float32)
        f2 = lax.dot_general(a2, wh, (((0,), (1,)), ((), ())),
                             preferred_element_type=jnp.float32)
        e = f1 + f2
        t = jnp.maximum(e, _ALPHA * e)
        m_raw = jnp.max(e, axis=1, keepdims=True)
        m = jnp.maximum(m_raw, _ALPHA * m_raw)
        p = jnp.exp(t - m) * adj
        s = jnp.sum(p, axis=1, keepdims=True)
        hp = jnp.dot(p, wh, preferred_element_type=jnp.float32) / s
        out_ref[:, h * _NHID:(h + 1) * _NHID] = jnp.where(
            hp > 0.0, hp, jnp.exp(hp) - 1.0)


@jax.jit
def kernel(x, adj, Ws, As):
    wh = pl.pallas_call(
        _wh_body,
        grid=(_NHEADS,),
        in_specs=[
            pl.BlockSpec((_N, _NFEAT), lambda h: (0, 0)),
            pl.BlockSpec((1, _NFEAT, _NHID), lambda h: (h, 0, 0)),
        ],
        out_specs=pl.BlockSpec((1, _N, _NHID), lambda h: (h, 0, 0)),
        out_shape=jax.ShapeDtypeStruct((_NHEADS, _N, _NHID), jnp.float32),
    )(x, Ws)

    out = pl.pallas_call(
        _gat_body,
        grid=(_N // _BLK,),
        in_specs=[
            pl.BlockSpec((_BLK, _N), lambda i: (i, 0)),
            pl.BlockSpec((_NHEADS, _N, _NHID), lambda i: (0, 0, 0)),
            pl.BlockSpec((_NHEADS, 2 * _NHID, 1), lambda i: (0, 0, 0)),
        ],
        out_specs=pl.BlockSpec((_BLK, _NHEADS * _NHID), lambda i: (i, 0)),
        out_shape=jax.ShapeDtypeStruct((_N, _NHEADS * _NHID), jnp.float32),
    )(adj, wh, As)
    return out


# speedup vs baseline: 2.7738x; 1.3370x over previous
import functools

import jax
import jax.numpy as jnp
from jax import lax
from jax.experimental import pallas as pl
from jax.experimental.pallas import tpu as pltpu

_N = 4096
_NFEAT = 256
_NHID = 16
_NHEADS = 4
_ALPHA = 0.2
_BLK = 512


def _wh_body(x_ref, w_ref, wh_ref):
    wh = jnp.dot(x_ref[...], w_ref[0], preferred_element_type=jnp.float32)
    wh_ref[0] = jnp.concatenate(
        [wh, jnp.ones((_N, 1), jnp.float32)], axis=1)


def _gat_body(adj_ref, wh_ref, a_ref, out_ref):
    i = pl.program_id(0)
    adj = adj_ref[...]
    for h in range(_NHEADS):
        wh = wh_ref[h]
        whb = wh_ref[h, pl.ds(i * _BLK, _BLK), :_NHID]
        a1 = a_ref[h, :_NHID, :]
        a2 = a_ref[h, _NHID:, :]
        f1 = lax.dot_general(whb, a1, (((1,), (0,)), ((), ())),
                             preferred_element_type=jnp.float32)
        f2 = lax.dot_general(a2, wh[:, :_NHID], (((0,), (1,)), ((), ())),
                             preferred_element_type=jnp.float32)
        m_raw = f1 + jnp.max(f2)
        m = jnp.maximum(m_raw, _ALPHA * m_raw)
        g1 = f1 - m
        c = (_ALPHA - 1.0) * m
        x = g1 + f2
        y = jnp.maximum(x, _ALPHA * x + c)
        p = jnp.exp(y) * adj
        hp_aug = jnp.dot(p, wh, preferred_element_type=jnp.float32)
        hp = hp_aug[:, :_NHID] / hp_aug[:, _NHID:]
        out_ref[:, h * _NHID:(h + 1) * _NHID] = jnp.where(
            hp > 0.0, hp, jnp.exp(hp) - 1.0)


@jax.jit
def kernel(x, adj, Ws, As):
    wh = pl.pallas_call(
        _wh_body,
        grid=(_NHEADS,),
        in_specs=[
            pl.BlockSpec((_N, _NFEAT), lambda h: (0, 0)),
            pl.BlockSpec((1, _NFEAT, _NHID), lambda h: (h, 0, 0)),
        ],
        out_specs=pl.BlockSpec((1, _N, _NHID + 1), lambda h: (h, 0, 0)),
        out_shape=jax.ShapeDtypeStruct((_NHEADS, _N, _NHID + 1), jnp.float32),
    )(x, Ws)

    out = pl.pallas_call(
        _gat_body,
        grid=(_N // _BLK,),
        in_specs=[
            pl.BlockSpec((_BLK, _N), lambda i: (i, 0)),
            pl.BlockSpec((_NHEADS, _N, _NHID + 1), lambda i: (0, 0, 0)),
            pl.BlockSpec((_NHEADS, 2 * _NHID, 1), lambda i: (0, 0, 0)),
        ],
        out_specs=pl.BlockSpec((_BLK, _NHEADS * _NHID), lambda i: (i, 0)),
        out_shape=jax.ShapeDtypeStruct((_N, _NHEADS * _NHID), jnp.float32),
    )(adj, wh, As)
    return out


# speedup vs baseline: 3.0061x; 1.0837x over previous
import functools

import jax
import jax.numpy as jnp
from jax import lax
from jax.experimental import pallas as pl
from jax.experimental.pallas import tpu as pltpu

_N = 4096
_NFEAT = 256
_NHID = 16
_NHEADS = 4
_ALPHA = 0.2
_BLK = 512


def _wh_body(x_ref, w_ref, wh_ref):
    wh = jnp.dot(x_ref[...], w_ref[0], preferred_element_type=jnp.float32)
    wh_ref[0] = jnp.concatenate(
        [wh, jnp.ones((_N, 1), jnp.float32)], axis=1)


def _gat_body(adj_ref, wh_ref, a_ref, out_ref):
    i = pl.program_id(0)
    adj = adj_ref[...]
    for h in range(_NHEADS):
        wh = wh_ref[h]
        whb = wh_ref[h, pl.ds(i * _BLK, _BLK), :_NHID]
        a1 = a_ref[h, :_NHID, :]
        a2 = a_ref[h, _NHID:, :]
        f1 = lax.dot_general(whb, a1, (((1,), (0,)), ((), ())),
                             preferred_element_type=jnp.float32)
        f2 = lax.dot_general(a2, wh[:, :_NHID], (((0,), (1,)), ((), ())),
                             preferred_element_type=jnp.float32)
        m_raw = f1 + jnp.max(f2)
        m = jnp.maximum(m_raw, _ALPHA * m_raw)
        l2e = 1.4426950408889634
        g1 = (f1 - m) * l2e
        c = (_ALPHA - 1.0) * m * l2e
        x = g1 + f2 * l2e
        y = jnp.maximum(x, _ALPHA * x + c)
        p = jnp.exp2(y) * adj
        hp_aug = jnp.dot(p, wh, preferred_element_type=jnp.float32)
        hp = hp_aug[:, :_NHID] / hp_aug[:, _NHID:]
        out_ref[:, h * _NHID:(h + 1) * _NHID] = jnp.where(
            hp > 0.0, hp, jnp.exp(hp) - 1.0)


@jax.jit
def kernel(x, adj, Ws, As):
    wh = pl.pallas_call(
        _wh_body,
        grid=(_NHEADS,),
        in_specs=[
            pl.BlockSpec((_N, _NFEAT), lambda h: (0, 0)),
            pl.BlockSpec((1, _NFEAT, _NHID), lambda h: (h, 0, 0)),
        ],
        out_specs=pl.BlockSpec((1, _N, _NHID + 1), lambda h: (h, 0, 0)),
        out_shape=jax.ShapeDtypeStruct((_NHEADS, _N, _NHID + 1), jnp.float32),
    )(x, Ws)

    out = pl.pallas_call(
        _gat_body,
        grid=(_N // _BLK,),
        in_specs=[
            pl.BlockSpec((_BLK, _N), lambda i: (i, 0)),
            pl.BlockSpec((_NHEADS, _N, _NHID + 1), lambda i: (0, 0, 0)),
            pl.BlockSpec((_NHEADS, 2 * _NHID, 1), lambda i: (0, 0, 0)),
        ],
        out_specs=pl.BlockSpec((_BLK, _NHEADS * _NHID), lambda i: (i, 0)),
        out_shape=jax.ShapeDtypeStruct((_N, _NHEADS * _NHID), jnp.float32),
    )(adj, wh, As)
    return out


# speedup vs baseline: 3.5824x; 1.1917x over previous
import functools

import jax
import jax.numpy as jnp
from jax import lax
from jax.experimental import pallas as pl
from jax.experimental.pallas import tpu as pltpu

_N = 4096
_NFEAT = 256
_NHID = 16
_NHEADS = 4
_ALPHA = 0.2
_BLK = 512


def _wh_body(x_ref, w_ref, wh_ref):
    wh = jnp.dot(x_ref[...], w_ref[0], preferred_element_type=jnp.float32)
    wh_ref[0] = jnp.concatenate(
        [wh, jnp.ones((_N, 1), jnp.float32)], axis=1)


def _gat_body(adj_ref, wh_ref, a_ref, out_ref):
    i = pl.program_id(0)
    adj = adj_ref[...].astype(jnp.bfloat16)
    for h in range(_NHEADS):
        wh = wh_ref[h]
        whb = wh_ref[h, pl.ds(i * _BLK, _BLK), :_NHID]
        a1 = a_ref[h, :_NHID, :]
        a2 = a_ref[h, _NHID:, :]
        f1 = lax.dot_general(whb, a1, (((1,), (0,)), ((), ())),
                             preferred_element_type=jnp.float32)
        f2 = lax.dot_general(a2, wh[:, :_NHID], (((0,), (1,)), ((), ())),
                             preferred_element_type=jnp.float32)
        m_raw = f1 + jnp.max(f2)
        m = jnp.maximum(m_raw, _ALPHA * m_raw)
        l2e = 1.4426950408889634
        g1 = ((f1 - m) * l2e).astype(jnp.bfloat16)
        c = ((_ALPHA - 1.0) * m * l2e).astype(jnp.bfloat16)
        f2b = (f2 * l2e).astype(jnp.bfloat16)
        x = g1 + f2b
        y = jnp.maximum(x, jnp.bfloat16(_ALPHA) * x + c)
        p = jnp.exp2(y) * adj
        hp_aug = jnp.dot(p, wh.astype(jnp.bfloat16),
                         preferred_element_type=jnp.float32)
        hp = hp_aug[:, :_NHID] / hp_aug[:, _NHID:]
        out_ref[:, h * _NHID:(h + 1) * _NHID] = jnp.where(
            hp > 0.0, hp, jnp.exp(hp) - 1.0)


@jax.jit
def kernel(x, adj, Ws, As):
    wh = pl.pallas_call(
        _wh_body,
        grid=(_NHEADS,),
        in_specs=[
            pl.BlockSpec((_N, _NFEAT), lambda h: (0, 0)),
            pl.BlockSpec((1, _NFEAT, _NHID), lambda h: (h, 0, 0)),
        ],
        out_specs=pl.BlockSpec((1, _N, _NHID + 1), lambda h: (h, 0, 0)),
        out_shape=jax.ShapeDtypeStruct((_NHEADS, _N, _NHID + 1), jnp.float32),
    )(x, Ws)

    out = pl.pallas_call(
        _gat_body,
        grid=(_N // _BLK,),
        in_specs=[
            pl.BlockSpec((_BLK, _N), lambda i: (i, 0)),
            pl.BlockSpec((_NHEADS, _N, _NHID + 1), lambda i: (0, 0, 0)),
            pl.BlockSpec((_NHEADS, 2 * _NHID, 1), lambda i: (0, 0, 0)),
        ],
        out_specs=pl.BlockSpec((_BLK, _NHEADS * _NHID), lambda i: (i, 0)),
        out_shape=jax.ShapeDtypeStruct((_N, _NHEADS * _NHID), jnp.float32),
    )(adj, wh, As)
    return out
